# baseline (speedup 1.0000x reference)
import jax
import jax.numpy as jnp
from jax import lax
from jax.experimental import pallas as pl
from jax.experimental.pallas import tpu as pltpu
from jax.experimental.pallas import tpu_sc as plsc

NN = 10000
EE = 160000
NGRP = 9
NW = 32
NV = 313
CHUNK = 128
NB = 1000
GRID = NN // NB
NTOT = NW * NV
BIGF = 3.0e38
EB = 2048
EPAD = 160 * 1024

_SC_PARAMS = pltpu.CompilerParams(use_tc_tiling_on_sc=False,
                                  needs_layout_passes=False)


def _sc_gather_call(xt, sdst, ssrc, bnds, F):
    mesh = plsc.VectorSubcoreMesh(core_axis_name="c", subcore_axis_name="s")
    out_type = [jax.ShapeDtypeStruct((EPAD, F), jnp.float32)] * 2
    scratch = [
        pltpu.VMEM((CHUNK,), jnp.int32),
        pltpu.VMEM((CHUNK,), jnp.int32),
        pltpu.VMEM((CHUNK, F), jnp.float32),
        pltpu.VMEM((CHUNK, F), jnp.float32),
        pltpu.VMEM((48,), jnp.int32),
        pltpu.SemaphoreType.DMA,
        pltpu.SemaphoreType.DMA,
        pltpu.SemaphoreType.DMA,
        pltpu.SemaphoreType.DMA,
    ]

    def body(xt_h, dst_h, src_h, bnds_h, od_h, os_h,
             v_dst, v_src, v_rd, v_rs, v_bnds, s1, s2, s3, s4):
        cid = lax.axis_index("c")
        sid = lax.axis_index("s")
        wid = sid * 2 + cid
        pltpu.sync_copy(bnds_h, v_bnds)
        bidx = lax.iota(jnp.int32, 16) + wid
        bwin = plsc.load_gather(v_bnds, [bidx])
        b0 = bwin[0]
        b1 = bwin[1]
        b0a = (b0 // 8) * 8
        nch = (b1 - b0a + CHUNK - 1) // CHUNK

        def chunk_body(ci, carry):
            base = b0a + ci * CHUNK
            pltpu.sync_copy(dst_h.at[pl.ds(base, CHUNK)], v_dst)
            pltpu.sync_copy(src_h.at[pl.ds(base, CHUNK)], v_src)
            cp1 = pltpu.async_copy(xt_h.at[v_dst], v_rd, s1)
            cp2 = pltpu.async_copy(xt_h.at[v_src], v_rs, s2)
            cp1.wait()
            cp2.wait()
            cp3 = pltpu.async_copy(v_rd, od_h.at[pl.ds(base, CHUNK)], s3)
            cp4 = pltpu.async_copy(v_rs, os_h.at[pl.ds(base, CHUNK)], s4)
            cp3.wait()
            cp4.wait()
            return carry
        lax.fori_loop(0, nch, chunk_body, 0)

    fn = pl.kernel(body, out_type=tuple(out_type), mesh=mesh,
                   scratch_types=tuple(scratch), compiler_params=_SC_PARAMS)
    return fn(xt, sdst, ssrc, bnds)


def _sc_stats_call(m_in, sdst, bnds, Fh, with_cnt):
    KF = Fh // 16
    mesh = plsc.VectorSubcoreMesh(core_axis_name="c", subcore_axis_name="s")
    out_type = [jax.ShapeDtypeStruct((NW, NV * Fh), jnp.float32)] * 4
    if with_cnt:
        out_type.append(jax.ShapeDtypeStruct((NW, NV * 16), jnp.float32))
    scratch = [pltpu.VMEM(((NV + 1) * Fh,), jnp.float32) for _ in range(4)]
    if with_cnt:
        scratch.append(pltpu.VMEM(((NV + 1) * 16,), jnp.float32))
    scratch += [
        pltpu.VMEM((CHUNK,), jnp.int32),
        pltpu.VMEM((CHUNK, Fh), jnp.float32),
        pltpu.VMEM((48,), jnp.int32),
        pltpu.SemaphoreType.DMA,
    ]
    ns = 5 if with_cnt else 4

    def body(m_h, dst_h, bnds_h, *rest):
        outs = rest[:ns]
        accs = rest[ns:2 * ns]
        v_dst, v_m, v_bnds, sem1 = rest[2 * ns:]
        cid = lax.axis_index("c")
        sid = lax.axis_index("s")
        wid = sid * 2 + cid
        v0 = wid * NV
        pltpu.sync_copy(bnds_h, v_bnds)
        bidx = lax.iota(jnp.int32, 16) + wid
        bwin = plsc.load_gather(v_bnds, [bidx])
        b0 = bwin[0]
        b1 = bwin[1]
        b0a = (b0 // 8) * 8
        nch = (b1 - b0a + CHUNK - 1) // CHUNK

        zv = jnp.zeros((16,), jnp.float32)
        lov = jnp.full((16,), -BIGF, jnp.float32)
        hiv = jnp.full((16,), BIGF, jnp.float32)
        ones = jnp.ones((16,), jnp.float32)

        def init_body(i, carry):
            off = i * 16
            accs[0][pl.ds(off, 16)] = zv
            accs[1][pl.ds(off, 16)] = zv
            accs[2][pl.ds(off, 16)] = hiv
            accs[3][pl.ds(off, 16)] = lov
            return carry
        lax.fori_loop(0, (NV + 1) * KF, init_body, 0)
        if with_cnt:
            def initc(i, carry):
                accs[4][pl.ds(i * 16, 16)] = zv
                return carry
            lax.fori_loop(0, NV + 1, initc, 0)

        def chunk_body(ci, carry):
            base = b0a + ci * CHUNK
            pltpu.sync_copy(dst_h.at[pl.ds(base, CHUNK)], v_dst)
            pltpu.sync_copy(m_h.at[pl.ds(base, CHUNK)], v_m)

            def group_body(q, carry2):
                e0 = q * 16
                dvec = v_dst[pl.ds(e0, 16)]
                for j in range(16):
                    g = base + e0 + j
                    valid = jnp.logical_and(g >= b0, g < b1)
                    dl = jnp.where(valid, dvec[j] - v0, NV)
                    off = dl * Fh
                    e = e0 + j
                    for k in range(KF):
                        col = k * 16
                        mk = v_m[e, pl.ds(col, 16)]
                        so = off + col
                        accs[0][pl.ds(so, 16)] = accs[0][pl.ds(so, 16)] + mk
                        accs[1][pl.ds(so, 16)] = (accs[1][pl.ds(so, 16)]
                                                  + mk * mk)
                        accs[2][pl.ds(so, 16)] = jnp.minimum(
                            accs[2][pl.ds(so, 16)], mk)
                        accs[3][pl.ds(so, 16)] = jnp.maximum(
                            accs[3][pl.ds(so, 16)], mk)
                    if with_cnt:
                        co = dl * 16
                        accs[4][pl.ds(co, 16)] = (accs[4][pl.ds(co, 16)]
                                                  + ones)
                return carry2
            lax.fori_loop(0, CHUNK // 16, group_body, 0)
            return carry
        lax.fori_loop(0, nch, chunk_body, 0)
        for j in range(4):
            pltpu.sync_copy(accs[j].at[pl.ds(0, NV * Fh)], outs[j].at[wid])
        if with_cnt:
            pltpu.sync_copy(accs[4].at[pl.ds(0, NV * 16)], outs[4].at[wid])

    fn = pl.kernel(body, out_type=tuple(out_type), mesh=mesh,
                   scratch_types=tuple(scratch), compiler_params=_SC_PARAMS)
    return fn(m_in, sdst, bnds)


def _full(shape):
    return pl.BlockSpec(shape, lambda i: tuple(0 for _ in shape))


def _rows(nb, f):
    return pl.BlockSpec((nb, f), lambda i: (i, 0))


def _m128_body(xd_ref, xs_ref, ea_ref, eW_ref, eb_ref, pW_ref, pb_ref,
               lo_ref, hi_ref):
    e = jnp.dot(ea_ref[...], eW_ref[...],
                preferred_element_type=jnp.float32) + eb_ref[...]
    h = jnp.concatenate([xd_ref[...], xs_ref[...], e], axis=1)
    m = jnp.dot(h, pW_ref[...], preferred_element_type=jnp.float32) \
        + pb_ref[...]
    lo_ref[...] = m[:, 0:64]
    hi_ref[...] = m[:, 64:128]


def _m128_call(xd_g, xs_g, sea_p, eW, eb, pW, pb):
    return pl.pallas_call(
        _m128_body,
        grid=(EPAD // EB,),
        in_specs=[_rows(EB, 128), _rows(EB, 128), _rows(EB, 4),
                  _full((4, 128)), _full((1, 128)), _full((384, 128)),
                  _full((1, 128))],
        out_specs=[_rows(EB, 64), _rows(EB, 64)],
        out_shape=[jax.ShapeDtypeStruct((EPAD, 64), jnp.float32)] * 2,
    )(xd_g, xs_g, sea_p, eW, eb.reshape(1, 128), pW, pb.reshape(1, 128))


def _m16_body(xd_ref, xs_ref, ea_ref, eW_ref, eb_ref, pW_ref, pb_ref, o_ref):
    e = jnp.dot(ea_ref[...], eW_ref[...],
                preferred_element_type=jnp.float32) + eb_ref[...]
    h = jnp.concatenate([xd_ref[..., 0:4], xs_ref[..., 0:4], e], axis=1)
    o_ref[...] = jnp.dot(h, pW_ref[...],
                         preferred_element_type=jnp.float32) + pb_ref[...]


def _m16_call(xd_g, xs_g, sea_p, eW, eb, pW_pad, pb_pad):
    return pl.pallas_call(
        _m16_body,
        grid=(EPAD // EB,),
        in_specs=[_rows(EB, 16), _rows(EB, 16), _rows(EB, 4),
                  _full((4, 4)), _full((1, 4)), _full((12, 16)),
                  _full((1, 16))],
        out_specs=_rows(EB, 16),
        out_shape=jax.ShapeDtypeStruct((EPAD, 16), jnp.float32),
    )(xd_g, xs_g, sea_p, eW, eb.reshape(1, 4), pW_pad, pb_pad.reshape(1, 16))


def _deg_body(cnt_ref, dcols_ref, sums_ref):
    i = pl.program_id(0)
    c = cnt_ref[...]
    deg = jnp.maximum(c, 1.0)
    logd = jnp.log(deg + 1.0)
    has = (c > 0).astype(jnp.float32)
    dcols_ref[...] = jnp.concatenate(
        [deg, logd, has, jnp.zeros_like(c)], axis=1)
    blk = jnp.concatenate(
        [jnp.sum(deg).reshape(1, 1), jnp.sum(logd).reshape(1, 1),
         jnp.zeros((1, 6), jnp.float32)], axis=1)

    @pl.when(i == 0)
    def _():
        sums_ref[...] = jnp.zeros_like(sums_ref)
    sums_ref[...] += blk


def _deg_call(cnt):
    return pl.pallas_call(
        _deg_body,
        grid=(GRID,),
        in_specs=[_rows(NB, 1)],
        out_specs=[_rows(NB, 4), _full((1, 8))],
        out_shape=[jax.ShapeDtypeStruct((NN, 4), jnp.float32),
                   jax.ShapeDtypeStruct((1, 8), jnp.float32)],
    )(cnt)


def _make_conv_body(f, nseg):
    def body(*refs):
        srefs = refs[:4 * nseg]
        (x_ref, dc_ref, sums_ref, oW_ref, ob_ref, lW_ref,
         lb_ref, o_ref, bn_ref) = refs[4 * nseg:]
        i = pl.program_id(0)
        dc = dc_ref[...]
        deg = dc[:, 0:1]
        logd = dc[:, 1:2]
        has = dc[:, 2:3]
        avg_lin = sums_ref[0, 0] / NN
        avg_log = sums_ref[0, 1] / NN

        def cat(j):
            v = jnp.concatenate([srefs[j * nseg + s][...]
                                 for s in range(nseg)], axis=1) if nseg > 1 \
                else srefs[j][...]
            return v[:, :f]
        ssum = cat(0)
        ssq = cat(1)
        mean = ssum / deg
        meansq = ssq / deg
        std = jnp.sqrt(jnp.maximum(meansq - mean * mean, 0.0) + 1e-5)
        mn = jnp.where(has > 0, cat(2), 0.0)
        mx = jnp.where(has > 0, cat(3), 0.0)
        agg = jnp.concatenate([mean, mn, mx, std], axis=1)
        s1 = logd / avg_log
        s2 = avg_log / logd
        s3 = deg / avg_lin
        h2 = jnp.concatenate([x_ref[...], agg, agg * s1, agg * s2, agg * s3],
                             axis=1)
        out = jnp.dot(h2, oW_ref[...],
                      preferred_element_type=jnp.float32) + ob_ref[...]
        out = jnp.dot(out, lW_ref[...],
                      preferred_element_type=jnp.float32) + lb_ref[...]
        o_ref[...] = out

        @pl.when(i == 0)
        def _():
            bn_ref[...] = jnp.zeros_like(bn_ref)
        bn_ref[...] += jnp.sum(out, 0, keepdims=True)
    return body


def _conv_call(stat_arrays, xc, dcols, sums, oW, ob, lW, lb, f, nseg):
    fp = stat_arrays[0].shape[1]
    in_specs = ([_rows(NB, fp)] * (4 * nseg)
                + [_rows(NB, xc.shape[1]), _rows(NB, 4), _full((1, 8)),
                   _full(oW.shape), _full((1, 128)),
                   _full((128, 128)), _full((1, 128))])
    return pl.pallas_call(
        _make_conv_body(f, nseg),
        grid=(GRID,),
        in_specs=in_specs,
        out_specs=[_rows(NB, 128), _full((1, 128))],
        out_shape=[jax.ShapeDtypeStruct((NN, 128), jnp.float32),
                   jax.ShapeDtypeStruct((1, 128), jnp.float32)],
    )(*stat_arrays, xc, dcols, sums, oW, ob.reshape(1, 128), lW,
      lb.reshape(1, 128))


def _var_body(o_ref, bn_ref, v_ref):
    i = pl.program_id(0)
    mu = bn_ref[...] / NN
    d = o_ref[...] - mu

    @pl.when(i == 0)
    def _():
        v_ref[...] = jnp.zeros_like(v_ref)
    v_ref[...] += jnp.sum(d * d, 0, keepdims=True)


def _var_call(out2, bn):
    return pl.pallas_call(
        _var_body,
        grid=(GRID,),
        in_specs=[_rows(NB, 128), _full((1, 128))],
        out_specs=_full((1, 128)),
        out_shape=jax.ShapeDtypeStruct((1, 128), jnp.float32),
    )(out2, bn)


def _fin_body(o_ref, bn_ref, vs_ref, g_ref, b_ref, xn_ref):
    mu = bn_ref[...] / NN
    var = vs_ref[...] / NN
    o = (o_ref[...] - mu) / jnp.sqrt(var + 1e-5) * g_ref[...] + b_ref[...]
    xn_ref[...] = jnp.where(o >= 0, o, 0.01 * o)


def _fin_call(out2, bn, vs, g, b):
    return pl.pallas_call(
        _fin_body,
        grid=(GRID,),
        in_specs=[_rows(NB, 128), _full((1, 128)), _full((1, 128)),
                  _full((1, 128)), _full((1, 128))],
        out_specs=_rows(NB, 128),
        out_shape=jax.ShapeDtypeStruct((NN, 128), jnp.float32),
    )(out2, bn, vs, g.reshape(1, 128), b.reshape(1, 128))


def _pool_body(x_ref, b_ref, w1_ref, b1_ref, w2_ref, b2_ref, w3_ref, b3_ref,
               out_ref, s_sum, s_max, s_cnt):
    i = pl.program_id(0)

    @pl.when(i == 0)
    def _():
        s_sum[...] = jnp.zeros_like(s_sum)
        s_max[...] = jnp.full_like(s_max, -BIGF)
        s_cnt[...] = jnp.zeros_like(s_cnt)

    xb = x_ref[...]
    bb = b_ref[...]
    for g in range(NGRP):
        mask = bb == g
        s = jnp.sum(jnp.where(mask, xb, 0.0), axis=0, keepdims=True)
        mx = jnp.max(jnp.where(mask, xb, -BIGF), axis=0, keepdims=True)
        cg = jnp.sum(mask.astype(jnp.float32))
        s_sum[g:g + 1, :] += s
        s_max[g:g + 1, :] = jnp.maximum(s_max[g:g + 1, :], mx)
        s_cnt[g:g + 1, :] += jnp.full((1, 128), 1.0, jnp.float32) * cg

    @pl.when(i == GRID - 1)
    def _():
        cnt = s_cnt[...]
        x1 = s_sum[...] / jnp.maximum(cnt, 1.0)
        x2 = jnp.where(cnt > 0, s_max[...], 0.0)
        z = jnp.concatenate([x1, x2], axis=1)
        z = jnp.dot(z, w1_ref[...],
                    preferred_element_type=jnp.float32) + b1_ref[...]
        z = jnp.where(z >= 0, z, 0.01 * z)
        z = jnp.dot(z, w2_ref[...],
                    preferred_element_type=jnp.float32) + b2_ref[...]
        z = jnp.where(z >= 0, z, 0.01 * z)
        o = jnp.dot(z, w3_ref[...],
                    preferred_element_type=jnp.float32) + b3_ref[...]
        out_ref[...] = o[:NGRP, :]


def _pool_call(xc, batch2, l1W, l1b, l2W, l2b, l3W, l3b):
    return pl.pallas_call(
        _pool_body,
        grid=(GRID,),
        in_specs=[_rows(NB, 128), _rows(NB, 1), _full((256, 128)),
                  _full((1, 128)), _full((128, 64)), _full((1, 64)),
                  _full((64, 1)), _full((1, 1))],
        out_specs=_full((NGRP, 1)),
        out_shape=jax.ShapeDtypeStruct((NGRP, 1), jnp.float32),
        scratch_shapes=[pltpu.VMEM((16, 128), jnp.float32),
                        pltpu.VMEM((16, 128), jnp.float32),
                        pltpu.VMEM((16, 128), jnp.float32)],
    )(xc, batch2, l1W, l1b.reshape(1, 128), l2W, l2b.reshape(1, 64), l3W,
      l3b.reshape(1, 1))


def kernel(x, edge_index, edge_attr, batch, params):
    src = edge_index[0]
    dst = edge_index[1]
    perm = jnp.argsort(dst)
    sdst = dst[perm]
    ssrc = src[perm]
    sea = edge_attr[perm]
    pad = EPAD - EE
    sdst_p = jnp.concatenate([sdst, jnp.zeros((pad,), jnp.int32)])
    ssrc_p = jnp.concatenate([ssrc, jnp.zeros((pad,), jnp.int32)])
    sea_p = jnp.concatenate([sea, jnp.zeros((pad, 4), jnp.float32)])
    ranges = jnp.minimum(jnp.arange(NW + 1, dtype=jnp.int32) * NV, NN)
    bnds = jnp.searchsorted(sdst, ranges, side='left').astype(jnp.int32)
    bnds_p = jnp.concatenate([bnds, jnp.full((48 - NW - 1,), EE, jnp.int32)])
    convs = params['convs']

    p0 = convs[0]
    xpad = jnp.pad(x, ((0, 0), (0, 12)))
    xd_g, xs_g = _sc_gather_call(xpad, sdst_p, ssrc_p, bnds_p, 16)
    pW_pad = jnp.pad(p0['pW'], ((0, 0), (0, 12)))
    pb_pad = jnp.pad(p0['pb'], (0, 12))
    m16 = _m16_call(xd_g, xs_g, sea_p, p0['eW'], p0['eb'], pW_pad, pb_pad)
    ssum, ssq, smn, smx, scnt = _sc_stats_call(m16, sdst_p, bnds_p, 16, True)
    cnt = scnt.reshape(NTOT, 16)[:NN, 0:1]
    dcols, sums = _deg_call(cnt)
    stat_arrays = [ssum.reshape(NTOT, 16), ssq.reshape(NTOT, 16),
                   smn.reshape(NTOT, 16), smx.reshape(NTOT, 16)]
    nseg = 1

    xc = x
    for li in range(6):
        p = convs[li]
        f = 4 if li == 0 else 128
        out2, bn = _conv_call(stat_arrays, xc, dcols, sums, p['oW'],
                              p['ob'], p['lW'], p['lb'], f, nseg)
        vs = _var_call(out2, bn)
        xc = _fin_call(out2, bn, vs, p['bn_g'], p['bn_b'])
        if li < 5:
            pn = convs[li + 1]
            xd_g, xs_g = _sc_gather_call(xc, sdst_p, ssrc_p, bnds_p, 128)
            m_lo, m_hi = _m128_call(xd_g, xs_g, sea_p, pn['eW'], pn['eb'],
                                    pn['pW'], pn['pb'])
            r_lo = _sc_stats_call(m_lo, sdst_p, bnds_p, 64, False)
            r_hi = _sc_stats_call(m_hi, sdst_p, bnds_p, 64, False)
            stat_arrays = []
            for a, b in zip(r_lo, r_hi):
                stat_arrays.append(a.reshape(NTOT, 64))
                stat_arrays.append(b.reshape(NTOT, 64))
            nseg = 2

    batch2 = batch.reshape(NN, 1)
    return _pool_call(xc, batch2, params['l1W'], params['l1b'],
                      params['l2W'], params['l2b'], params['l3W'],
                      params['l3b'])


# speedup vs baseline: 1.2010x; 1.2010x over previous
import jax
import jax.numpy as jnp
from jax import lax
from jax.experimental import pallas as pl
from jax.experimental.pallas import tpu as pltpu
from jax.experimental.pallas import tpu_sc as plsc

NN = 10000
EE = 160000
NGRP = 9
NW = 32
NV = 313
CHUNK = 128
SUP = 1024
SPC = SUP // CHUNK
NB = 1000
GRID = NN // NB
NTOT = NW * NV
BIGF = 3.0e38
EB = 2048
EPAD = 160 * 1024

_SC_PARAMS = pltpu.CompilerParams(use_tc_tiling_on_sc=False,
                                  needs_layout_passes=False)


def _sc_gather_call(xt, sdst, ssrc, bnds, F):
    mesh = plsc.VectorSubcoreMesh(core_axis_name="c", subcore_axis_name="s")
    out_type = [jax.ShapeDtypeStruct((EPAD, F), jnp.float32)] * 2
    scratch = (
        [pltpu.VMEM((CHUNK,), jnp.int32)] * 2
        + [pltpu.VMEM((CHUNK, F), jnp.float32)] * 4
        + [pltpu.VMEM((48,), jnp.int32)]
        + [pltpu.SemaphoreType.DMA] * 8
    )

    def body(xt_h, dst_h, src_h, bnds_h, od_h, os_h,
             v_dst, v_src, v_rd0, v_rs0, v_rd1, v_rs1, v_bnds,
             g0, g1, g2, g3, w0, w1, w2, w3):
        cid = lax.axis_index("c")
        sid = lax.axis_index("s")
        wid = sid * 2 + cid
        pltpu.sync_copy(bnds_h, v_bnds)
        bidx = lax.iota(jnp.int32, 16) + wid
        bwin = plsc.load_gather(v_bnds, [bidx])
        b0 = bwin[0]
        b1 = bwin[1]
        b0a = (b0 // 8) * 8
        nch = (b1 - b0a + CHUNK - 1) // CHUNK
        npair = (nch + 1) // 2
        rd = (v_rd0, v_rd1)
        rs = (v_rs0, v_rs1)
        gsem = ((g0, g1), (g2, g3))
        wsem = ((w0, w1), (w2, w3))

        def pair_body(pi, carry):
            for par in range(2):
                ci = 2 * pi + par

                @pl.when(ci < nch)
                def _():
                    base = b0a + ci * CHUNK
                    pltpu.sync_copy(dst_h.at[pl.ds(base, CHUNK)], v_dst)
                    pltpu.sync_copy(src_h.at[pl.ds(base, CHUNK)], v_src)

                    @pl.when(ci >= 2)
                    def _():
                        pltpu.make_async_copy(
                            rd[par], od_h.at[pl.ds(base, CHUNK)],
                            wsem[par][0]).wait()
                        pltpu.make_async_copy(
                            rs[par], os_h.at[pl.ds(base, CHUNK)],
                            wsem[par][1]).wait()
                    cp1 = pltpu.async_copy(xt_h.at[v_dst], rd[par],
                                           gsem[par][0])
                    cp2 = pltpu.async_copy(xt_h.at[v_src], rs[par],
                                           gsem[par][1])
                    cp1.wait()
                    cp2.wait()
                    pltpu.async_copy(rd[par], od_h.at[pl.ds(base, CHUNK)],
                                     wsem[par][0])
                    pltpu.async_copy(rs[par], os_h.at[pl.ds(base, CHUNK)],
                                     wsem[par][1])
            return carry
        lax.fori_loop(0, npair, pair_body, 0)
        for par in range(2):
            @pl.when(nch > par)
            def _(par=par):
                pltpu.make_async_copy(rd[par], od_h.at[pl.ds(0, CHUNK)],
                                      wsem[par][0]).wait()
                pltpu.make_async_copy(rs[par], os_h.at[pl.ds(0, CHUNK)],
                                      wsem[par][1]).wait()

    fn = pl.kernel(body, out_type=tuple(out_type), mesh=mesh,
                   scratch_types=tuple(scratch), compiler_params=_SC_PARAMS)
    return fn(xt, sdst, ssrc, bnds)


def _sc_stats_call(m_in, sdst, bnds, Fh, with_cnt):
    KF = Fh // 16
    mesh = plsc.VectorSubcoreMesh(core_axis_name="c", subcore_axis_name="s")
    out_type = [jax.ShapeDtypeStruct((NW, NV * Fh), jnp.float32)] * 4
    if with_cnt:
        out_type.append(jax.ShapeDtypeStruct((NW, NV * 16), jnp.float32))
    scratch = [pltpu.VMEM(((NV + 1) * Fh,), jnp.float32) for _ in range(4)]
    if with_cnt:
        scratch.append(pltpu.VMEM(((NV + 1) * 16,), jnp.float32))
    scratch += [
        pltpu.VMEM((SUP,), jnp.int32),
        pltpu.VMEM((CHUNK, Fh), jnp.float32),
        pltpu.VMEM((CHUNK, Fh), jnp.float32),
        pltpu.VMEM((48,), jnp.int32),
        pltpu.SemaphoreType.DMA,
        pltpu.SemaphoreType.DMA,
    ]
    ns = 5 if with_cnt else 4

    def body(m_h, dst_h, bnds_h, *rest):
        outs = rest[:ns]
        accs = rest[ns:2 * ns]
        v_dst, v_m0, v_m1, v_bnds, ms0, ms1 = rest[2 * ns:]
        v_mb = (v_m0, v_m1)
        msem = (ms0, ms1)
        cid = lax.axis_index("c")
        sid = lax.axis_index("s")
        wid = sid * 2 + cid
        v0 = wid * NV
        pltpu.sync_copy(bnds_h, v_bnds)
        bidx = lax.iota(jnp.int32, 16) + wid
        bwin = plsc.load_gather(v_bnds, [bidx])
        b0 = bwin[0]
        b1 = bwin[1]
        b0a = (b0 // 8) * 8
        nch = (b1 - b0a + CHUNK - 1) // CHUNK
        nsup = (nch + SPC - 1) // SPC

        zv = jnp.zeros((16,), jnp.float32)
        lov = jnp.full((16,), -BIGF, jnp.float32)
        hiv = jnp.full((16,), BIGF, jnp.float32)
        ones = jnp.ones((16,), jnp.float32)

        def init_body(i, carry):
            off = i * 16
            accs[0][pl.ds(off, 16)] = zv
            accs[1][pl.ds(off, 16)] = zv
            accs[2][pl.ds(off, 16)] = hiv
            accs[3][pl.ds(off, 16)] = lov
            return carry
        lax.fori_loop(0, (NV + 1) * KF, init_body, 0)
        if with_cnt:
            def initc(i, carry):
                accs[4][pl.ds(i * 16, 16)] = zv
                return carry
            lax.fori_loop(0, NV + 1, initc, 0)

        def sup_body(si, carry):
            sbase = b0a + si * SUP
            pltpu.sync_copy(dst_h.at[pl.ds(sbase, SUP)], v_dst)
            pltpu.async_copy(m_h.at[pl.ds(sbase, CHUNK)], v_mb[0], msem[0])
            for c in range(SPC):
                cglob = si * SPC + c
                base = sbase + c * CHUNK

                @pl.when(cglob < nch)
                def _(c=c, base=base, cglob=cglob):
                    pltpu.make_async_copy(
                        m_h.at[pl.ds(base, CHUNK)], v_mb[c % 2],
                        msem[c % 2]).wait()
                    if c < SPC - 1:
                        @pl.when(cglob + 1 < nch)
                        def _():
                            pltpu.async_copy(
                                m_h.at[pl.ds(base + CHUNK, CHUNK)],
                                v_mb[(c + 1) % 2], msem[(c + 1) % 2])
                    v_m = v_mb[c % 2]

                    def group_body(q, carry2):
                        e0 = q * 16
                        dvec = v_dst[pl.ds(c * CHUNK + e0, 16)]
                        for j in range(16):
                            g = base + e0 + j
                            valid = jnp.logical_and(g >= b0, g < b1)
                            dl = jnp.where(valid, dvec[j] - v0, NV)
                            off = dl * Fh
                            e = e0 + j
                            for k in range(KF):
                                col = k * 16
                                mk = v_m[e, pl.ds(col, 16)]
                                so = off + col
                                plsc.addupdate(accs[0].at[pl.ds(so, 16)], mk)
                                plsc.addupdate(accs[1].at[pl.ds(so, 16)],
                                               mk * mk)
                                accs[2][pl.ds(so, 16)] = jnp.minimum(
                                    accs[2][pl.ds(so, 16)], mk)
                                accs[3][pl.ds(so, 16)] = jnp.maximum(
                                    accs[3][pl.ds(so, 16)], mk)
                            if with_cnt:
                                co = dl * 16
                                plsc.addupdate(accs[4].at[pl.ds(co, 16)],
                                               ones)
                        return carry2
                    lax.fori_loop(0, CHUNK // 16, group_body, 0)
            return carry
        lax.fori_loop(0, nsup, sup_body, 0)
        for j in range(4):
            pltpu.sync_copy(accs[j].at[pl.ds(0, NV * Fh)], outs[j].at[wid])
        if with_cnt:
            pltpu.sync_copy(accs[4].at[pl.ds(0, NV * 16)], outs[4].at[wid])

    fn = pl.kernel(body, out_type=tuple(out_type), mesh=mesh,
                   scratch_types=tuple(scratch), compiler_params=_SC_PARAMS)
    return fn(m_in, sdst, bnds)


def _full(shape):
    return pl.BlockSpec(shape, lambda i: tuple(0 for _ in shape))


def _rows(nb, f):
    return pl.BlockSpec((nb, f), lambda i: (i, 0))


def _m128_body(xd_ref, xs_ref, ea_ref, eW_ref, eb_ref, pW_ref, pb_ref,
               lo_ref, hi_ref):
    e = jnp.dot(ea_ref[...], eW_ref[...],
                preferred_element_type=jnp.float32) + eb_ref[...]
    h = jnp.concatenate([xd_ref[...], xs_ref[...], e], axis=1)
    m = jnp.dot(h, pW_ref[...], preferred_element_type=jnp.float32) \
        + pb_ref[...]
    lo_ref[...] = m[:, 0:64]
    hi_ref[...] = m[:, 64:128]


def _m128_call(xd_g, xs_g, sea_p, eW, eb, pW, pb):
    return pl.pallas_call(
        _m128_body,
        grid=(EPAD // EB,),
        in_specs=[_rows(EB, 128), _rows(EB, 128), _rows(EB, 4),
                  _full((4, 128)), _full((1, 128)), _full((384, 128)),
                  _full((1, 128))],
        out_specs=[_rows(EB, 64), _rows(EB, 64)],
        out_shape=[jax.ShapeDtypeStruct((EPAD, 64), jnp.float32)] * 2,
    )(xd_g, xs_g, sea_p, eW, eb.reshape(1, 128), pW, pb.reshape(1, 128))


def _m16_body(xd_ref, xs_ref, ea_ref, eW_ref, eb_ref, pW_ref, pb_ref, o_ref):
    e = jnp.dot(ea_ref[...], eW_ref[...],
                preferred_element_type=jnp.float32) + eb_ref[...]
    h = jnp.concatenate([xd_ref[..., 0:4], xs_ref[..., 0:4], e], axis=1)
    o_ref[...] = jnp.dot(h, pW_ref[...],
                         preferred_element_type=jnp.float32) + pb_ref[...]


def _m16_call(xd_g, xs_g, sea_p, eW, eb, pW_pad, pb_pad):
    return pl.pallas_call(
        _m16_body,
        grid=(EPAD // EB,),
        in_specs=[_rows(EB, 16), _rows(EB, 16), _rows(EB, 4),
                  _full((4, 4)), _full((1, 4)), _full((12, 16)),
                  _full((1, 16))],
        out_specs=_rows(EB, 16),
        out_shape=jax.ShapeDtypeStruct((EPAD, 16), jnp.float32),
    )(xd_g, xs_g, sea_p, eW, eb.reshape(1, 4), pW_pad, pb_pad.reshape(1, 16))


def _deg_body(cnt_ref, dcols_ref, sums_ref):
    i = pl.program_id(0)
    c = cnt_ref[...]
    deg = jnp.maximum(c, 1.0)
    logd = jnp.log(deg + 1.0)
    has = (c > 0).astype(jnp.float32)
    dcols_ref[...] = jnp.concatenate(
        [deg, logd, has, jnp.zeros_like(c)], axis=1)
    blk = jnp.concatenate(
        [jnp.sum(deg).reshape(1, 1), jnp.sum(logd).reshape(1, 1),
         jnp.zeros((1, 6), jnp.float32)], axis=1)

    @pl.when(i == 0)
    def _():
        sums_ref[...] = jnp.zeros_like(sums_ref)
    sums_ref[...] += blk


def _deg_call(cnt):
    return pl.pallas_call(
        _deg_body,
        grid=(GRID,),
        in_specs=[_rows(NB, 1)],
        out_specs=[_rows(NB, 4), _full((1, 8))],
        out_shape=[jax.ShapeDtypeStruct((NN, 4), jnp.float32),
                   jax.ShapeDtypeStruct((1, 8), jnp.float32)],
    )(cnt)


def _make_conv_body(f, nseg):
    def body(*refs):
        srefs = refs[:4 * nseg]
        (x_ref, dc_ref, sums_ref, oW_ref, ob_ref, lW_ref,
         lb_ref, o_ref, bn_ref) = refs[4 * nseg:]
        i = pl.program_id(0)
        dc = dc_ref[...]
        deg = dc[:, 0:1]
        logd = dc[:, 1:2]
        has = dc[:, 2:3]
        avg_lin = sums_ref[0, 0] / NN
        avg_log = sums_ref[0, 1] / NN

        def cat(j):
            v = jnp.concatenate([srefs[j * nseg + s][...]
                                 for s in range(nseg)], axis=1) if nseg > 1 \
                else srefs[j][...]
            return v[:, :f]
        ssum = cat(0)
        ssq = cat(1)
        mean = ssum / deg
        meansq = ssq / deg
        std = jnp.sqrt(jnp.maximum(meansq - mean * mean, 0.0) + 1e-5)
        mn = jnp.where(has > 0, cat(2), 0.0)
        mx = jnp.where(has > 0, cat(3), 0.0)
        agg = jnp.concatenate([mean, mn, mx, std], axis=1)
        s1 = logd / avg_log
        s2 = avg_log / logd
        s3 = deg / avg_lin
        h2 = jnp.concatenate([x_ref[...], agg, agg * s1, agg * s2, agg * s3],
                             axis=1)
        out = jnp.dot(h2, oW_ref[...],
                      preferred_element_type=jnp.float32) + ob_ref[...]
        out = jnp.dot(out, lW_ref[...],
                      preferred_element_type=jnp.float32) + lb_ref[...]
        o_ref[...] = out

        @pl.when(i == 0)
        def _():
            bn_ref[...] = jnp.zeros_like(bn_ref)
        bn_ref[...] += jnp.sum(out, 0, keepdims=True)
    return body


def _conv_call(stat_arrays, xc, dcols, sums, oW, ob, lW, lb, f, nseg):
    fp = stat_arrays[0].shape[1]
    in_specs = ([_rows(NB, fp)] * (4 * nseg)
                + [_rows(NB, xc.shape[1]), _rows(NB, 4), _full((1, 8)),
                   _full(oW.shape), _full((1, 128)),
                   _full((128, 128)), _full((1, 128))])
    return pl.pallas_call(
        _make_conv_body(f, nseg),
        grid=(GRID,),
        in_specs=in_specs,
        out_specs=[_rows(NB, 128), _full((1, 128))],
        out_shape=[jax.ShapeDtypeStruct((NN, 128), jnp.float32),
                   jax.ShapeDtypeStruct((1, 128), jnp.float32)],
    )(*stat_arrays, xc, dcols, sums, oW, ob.reshape(1, 128), lW,
      lb.reshape(1, 128))


def _var_body(o_ref, bn_ref, v_ref):
    i = pl.program_id(0)
    mu = bn_ref[...] / NN
    d = o_ref[...] - mu

    @pl.when(i == 0)
    def _():
        v_ref[...] = jnp.zeros_like(v_ref)
    v_ref[...] += jnp.sum(d * d, 0, keepdims=True)


def _var_call(out2, bn):
    return pl.pallas_call(
        _var_body,
        grid=(GRID,),
        in_specs=[_rows(NB, 128), _full((1, 128))],
        out_specs=_full((1, 128)),
        out_shape=jax.ShapeDtypeStruct((1, 128), jnp.float32),
    )(out2, bn)


def _fin_body(o_ref, bn_ref, vs_ref, g_ref, b_ref, xn_ref):
    mu = bn_ref[...] / NN
    var = vs_ref[...] / NN
    o = (o_ref[...] - mu) / jnp.sqrt(var + 1e-5) * g_ref[...] + b_ref[...]
    xn_ref[...] = jnp.where(o >= 0, o, 0.01 * o)


def _fin_call(out2, bn, vs, g, b):
    return pl.pallas_call(
        _fin_body,
        grid=(GRID,),
        in_specs=[_rows(NB, 128), _full((1, 128)), _full((1, 128)),
                  _full((1, 128)), _full((1, 128))],
        out_specs=_rows(NB, 128),
        out_shape=jax.ShapeDtypeStruct((NN, 128), jnp.float32),
    )(out2, bn, vs, g.reshape(1, 128), b.reshape(1, 128))


def _pool_body(x_ref, b_ref, w1_ref, b1_ref, w2_ref, b2_ref, w3_ref, b3_ref,
               out_ref, s_sum, s_max, s_cnt):
    i = pl.program_id(0)

    @pl.when(i == 0)
    def _():
        s_sum[...] = jnp.zeros_like(s_sum)
        s_max[...] = jnp.full_like(s_max, -BIGF)
        s_cnt[...] = jnp.zeros_like(s_cnt)

    xb = x_ref[...]
    bb = b_ref[...]
    for g in range(NGRP):
        mask = bb == g
        s = jnp.sum(jnp.where(mask, xb, 0.0), axis=0, keepdims=True)
        mx = jnp.max(jnp.where(mask, xb, -BIGF), axis=0, keepdims=True)
        cg = jnp.sum(mask.astype(jnp.float32))
        s_sum[g:g + 1, :] += s
        s_max[g:g + 1, :] = jnp.maximum(s_max[g:g + 1, :], mx)
        s_cnt[g:g + 1, :] += jnp.full((1, 128), 1.0, jnp.float32) * cg

    @pl.when(i == GRID - 1)
    def _():
        cnt = s_cnt[...]
        x1 = s_sum[...] / jnp.maximum(cnt, 1.0)
        x2 = jnp.where(cnt > 0, s_max[...], 0.0)
        z = jnp.concatenate([x1, x2], axis=1)
        z = jnp.dot(z, w1_ref[...],
                    preferred_element_type=jnp.float32) + b1_ref[...]
        z = jnp.where(z >= 0, z, 0.01 * z)
        z = jnp.dot(z, w2_ref[...],
                    preferred_element_type=jnp.float32) + b2_ref[...]
        z = jnp.where(z >= 0, z, 0.01 * z)
        o = jnp.dot(z, w3_ref[...],
                    preferred_element_type=jnp.float32) + b3_ref[...]
        out_ref[...] = o[:NGRP, :]


def _pool_call(xc, batch2, l1W, l1b, l2W, l2b, l3W, l3b):
    return pl.pallas_call(
        _pool_body,
        grid=(GRID,),
        in_specs=[_rows(NB, 128), _rows(NB, 1), _full((256, 128)),
                  _full((1, 128)), _full((128, 64)), _full((1, 64)),
                  _full((64, 1)), _full((1, 1))],
        out_specs=_full((NGRP, 1)),
        out_shape=jax.ShapeDtypeStruct((NGRP, 1), jnp.float32),
        scratch_shapes=[pltpu.VMEM((16, 128), jnp.float32),
                        pltpu.VMEM((16, 128), jnp.float32),
                        pltpu.VMEM((16, 128), jnp.float32)],
    )(xc, batch2, l1W, l1b.reshape(1, 128), l2W, l2b.reshape(1, 64), l3W,
      l3b.reshape(1, 1))


def kernel(x, edge_index, edge_attr, batch, params):
    src = edge_index[0]
    dst = edge_index[1]
    perm = jnp.argsort(dst)
    sdst = dst[perm]
    ssrc = src[perm]
    sea = edge_attr[perm]
    pad = EPAD - EE
    sdst_p = jnp.concatenate([sdst, jnp.zeros((pad,), jnp.int32)])
    ssrc_p = jnp.concatenate([ssrc, jnp.zeros((pad,), jnp.int32)])
    sea_p = jnp.concatenate([sea, jnp.zeros((pad, 4), jnp.float32)])
    ranges = jnp.minimum(jnp.arange(NW + 1, dtype=jnp.int32) * NV, NN)
    bnds = jnp.searchsorted(sdst, ranges, side='left').astype(jnp.int32)
    bnds_p = jnp.concatenate([bnds, jnp.full((48 - NW - 1,), EE, jnp.int32)])
    convs = params['convs']

    p0 = convs[0]
    xpad = jnp.pad(x, ((0, 0), (0, 12)))
    xd_g, xs_g = _sc_gather_call(xpad, sdst_p, ssrc_p, bnds_p, 16)
    pW_pad = jnp.pad(p0['pW'], ((0, 0), (0, 12)))
    pb_pad = jnp.pad(p0['pb'], (0, 12))
    m16 = _m16_call(xd_g, xs_g, sea_p, p0['eW'], p0['eb'], pW_pad, pb_pad)
    ssum, ssq, smn, smx, scnt = _sc_stats_call(m16, sdst_p, bnds_p, 16, True)
    cnt = scnt.reshape(NTOT, 16)[:NN, 0:1]
    dcols, sums = _deg_call(cnt)
    stat_arrays = [ssum.reshape(NTOT, 16), ssq.reshape(NTOT, 16),
                   smn.reshape(NTOT, 16), smx.reshape(NTOT, 16)]
    nseg = 1

    xc = x
    for li in range(6):
        p = convs[li]
        f = 4 if li == 0 else 128
        out2, bn = _conv_call(stat_arrays, xc, dcols, sums, p['oW'],
                              p['ob'], p['lW'], p['lb'], f, nseg)
        vs = _var_call(out2, bn)
        xc = _fin_call(out2, bn, vs, p['bn_g'], p['bn_b'])
        if li < 5:
            pn = convs[li + 1]
            xd_g, xs_g = _sc_gather_call(xc, sdst_p, ssrc_p, bnds_p, 128)
            m_lo, m_hi = _m128_call(xd_g, xs_g, sea_p, pn['eW'], pn['eb'],
                                    pn['pW'], pn['pb'])
            r_lo = _sc_stats_call(m_lo, sdst_p, bnds_p, 64, False)
            r_hi = _sc_stats_call(m_hi, sdst_p, bnds_p, 64, False)
            stat_arrays = []
            for a, b in zip(r_lo, r_hi):
                stat_arrays.append(a.reshape(NTOT, 64))
                stat_arrays.append(b.reshape(NTOT, 64))
            nseg = 2

    batch2 = batch.reshape(NN, 1)
    return _pool_call(xc, batch2, params['l1W'], params['l1b'],
                      params['l2W'], params['l2b'], params['l3W'],
                      params['l3b'])


# speedup vs baseline: 1.2338x; 1.0273x over previous
import jax
import jax.numpy as jnp
from jax import lax
from jax.experimental import pallas as pl
from jax.experimental.pallas import tpu as pltpu
from jax.experimental.pallas import tpu_sc as plsc

NN = 10000
EE = 160000
NGRP = 9
NW = 32
NV = 313
CHUNK = 128
SUP = 1024
SPC = SUP // CHUNK
NB = 1000
GRID = NN // NB
NTOT = NW * NV
BIGF = 3.0e38
EB = 2048
EPAD = 160 * 1024

_SC_PARAMS = pltpu.CompilerParams(use_tc_tiling_on_sc=False,
                                  needs_layout_passes=False)


def _sc_gather_call(xt, sdst, ssrc, bnds, F):
    mesh = plsc.VectorSubcoreMesh(core_axis_name="c", subcore_axis_name="s")
    out_type = [jax.ShapeDtypeStruct((EPAD, F), jnp.float32)] * 2
    scratch = (
        [pltpu.VMEM((SUP,), jnp.int32)] * 2
        + [pltpu.VMEM((CHUNK, F), jnp.float32)] * 4
        + [pltpu.VMEM((48,), jnp.int32)]
        + [pltpu.SemaphoreType.DMA] * 8
    )

    def body(xt_h, dst_h, src_h, bnds_h, od_h, os_h,
             v_dst, v_src, v_rd0, v_rs0, v_rd1, v_rs1, v_bnds,
             g0, g1, g2, g3, w0, w1, w2, w3):
        cid = lax.axis_index("c")
        sid = lax.axis_index("s")
        wid = sid * 2 + cid
        pltpu.sync_copy(bnds_h, v_bnds)
        bidx = lax.iota(jnp.int32, 16) + wid
        bwin = plsc.load_gather(v_bnds, [bidx])
        b0 = bwin[0]
        b1 = bwin[1]
        b0a = (b0 // 8) * 8
        nch = (b1 - b0a + CHUNK - 1) // CHUNK
        nsup = (nch + SPC - 1) // SPC
        rd = (v_rd0, v_rd1)
        rs = (v_rs0, v_rs1)
        gsem = ((g0, g1), (g2, g3))
        wsem = ((w0, w1), (w2, w3))

        def issue_gather(c, base):
            par = c % 2
            pltpu.async_copy(xt_h.at[v_dst.at[pl.ds(c * CHUNK, CHUNK)]],
                             rd[par], gsem[par][0])
            pltpu.async_copy(xt_h.at[v_src.at[pl.ds(c * CHUNK, CHUNK)]],
                             rs[par], gsem[par][1])

        def sup_body(si, carry):
            sbase = b0a + si * SUP
            pltpu.sync_copy(dst_h.at[pl.ds(sbase, SUP)], v_dst)
            pltpu.sync_copy(src_h.at[pl.ds(sbase, SUP)], v_src)
            issue_gather(0, sbase)
            for c in range(SPC):
                cglob = si * SPC + c
                base = sbase + c * CHUNK
                par = c % 2

                @pl.when(cglob < nch)
                def _(c=c, base=base, cglob=cglob, par=par):
                    pltpu.make_async_copy(
                        xt_h.at[v_dst.at[pl.ds(c * CHUNK, CHUNK)]],
                        rd[par], gsem[par][0]).wait()
                    pltpu.make_async_copy(
                        xt_h.at[v_src.at[pl.ds(c * CHUNK, CHUNK)]],
                        rs[par], gsem[par][1]).wait()
                    if c < SPC - 1:
                        @pl.when(cglob + 1 < nch)
                        def _():
                            if c >= 1:
                                pltpu.make_async_copy(
                                    rd[1 - par],
                                    od_h.at[pl.ds(base, CHUNK)],
                                    wsem[1 - par][0]).wait()
                                pltpu.make_async_copy(
                                    rs[1 - par],
                                    os_h.at[pl.ds(base, CHUNK)],
                                    wsem[1 - par][1]).wait()
                            issue_gather(c + 1, base + CHUNK)
                    pltpu.async_copy(rd[par], od_h.at[pl.ds(base, CHUNK)],
                                     wsem[par][0])
                    pltpu.async_copy(rs[par], os_h.at[pl.ds(base, CHUNK)],
                                     wsem[par][1])
            for par in range(2):
                @pl.when(si * SPC + par < nch)
                def _(par=par):
                    pltpu.make_async_copy(rd[par], od_h.at[pl.ds(0, CHUNK)],
                                          wsem[par][0]).wait()
                    pltpu.make_async_copy(rs[par], os_h.at[pl.ds(0, CHUNK)],
                                          wsem[par][1]).wait()
            return carry
        lax.fori_loop(0, nsup, sup_body, 0)

    fn = pl.kernel(body, out_type=tuple(out_type), mesh=mesh,
                   scratch_types=tuple(scratch), compiler_params=_SC_PARAMS)
    return fn(xt, sdst, ssrc, bnds)


def _sc_stats_call(m_in, sdst, bnds, Fh, with_cnt):
    KF = Fh // 16
    mesh = plsc.VectorSubcoreMesh(core_axis_name="c", subcore_axis_name="s")
    out_type = [jax.ShapeDtypeStruct((NW, NV * Fh), jnp.float32)] * 4
    if with_cnt:
        out_type.append(jax.ShapeDtypeStruct((NW, NV * 16), jnp.float32))
    scratch = [pltpu.VMEM(((NV + 1) * Fh,), jnp.float32) for _ in range(4)]
    if with_cnt:
        scratch.append(pltpu.VMEM(((NV + 1) * 16,), jnp.float32))
    scratch += [
        pltpu.VMEM((SUP,), jnp.int32),
        pltpu.VMEM((CHUNK, Fh), jnp.float32),
        pltpu.VMEM((CHUNK, Fh), jnp.float32),
        pltpu.VMEM((48,), jnp.int32),
        pltpu.SemaphoreType.DMA,
        pltpu.SemaphoreType.DMA,
    ]
    ns = 5 if with_cnt else 4

    def body(m_h, dst_h, bnds_h, *rest):
        outs = rest[:ns]
        accs = rest[ns:2 * ns]
        v_dst, v_m0, v_m1, v_bnds, ms0, ms1 = rest[2 * ns:]
        v_mb = (v_m0, v_m1)
        msem = (ms0, ms1)
        cid = lax.axis_index("c")
        sid = lax.axis_index("s")
        wid = sid * 2 + cid
        v0 = wid * NV
        pltpu.sync_copy(bnds_h, v_bnds)
        bidx = lax.iota(jnp.int32, 16) + wid
        bwin = plsc.load_gather(v_bnds, [bidx])
        b0 = bwin[0]
        b1 = bwin[1]
        b0a = (b0 // 8) * 8
        nch = (b1 - b0a + CHUNK - 1) // CHUNK
        nsup = (nch + SPC - 1) // SPC

        zv = jnp.zeros((16,), jnp.float32)
        lov = jnp.full((16,), -BIGF, jnp.float32)
        hiv = jnp.full((16,), BIGF, jnp.float32)
        ones = jnp.ones((16,), jnp.float32)

        def init_body(i, carry):
            off = i * 16
            accs[0][pl.ds(off, 16)] = zv
            accs[1][pl.ds(off, 16)] = zv
            accs[2][pl.ds(off, 16)] = hiv
            accs[3][pl.ds(off, 16)] = lov
            return carry
        lax.fori_loop(0, (NV + 1) * KF, init_body, 0)
        if with_cnt:
            def initc(i, carry):
                accs[4][pl.ds(i * 16, 16)] = zv
                return carry
            lax.fori_loop(0, NV + 1, initc, 0)

        def sup_body(si, carry):
            sbase = b0a + si * SUP
            pltpu.sync_copy(dst_h.at[pl.ds(sbase, SUP)], v_dst)
            pltpu.async_copy(m_h.at[pl.ds(sbase, CHUNK)], v_mb[0], msem[0])
            for c in range(SPC):
                cglob = si * SPC + c
                base = sbase + c * CHUNK

                @pl.when(cglob < nch)
                def _(c=c, base=base, cglob=cglob):
                    pltpu.make_async_copy(
                        m_h.at[pl.ds(base, CHUNK)], v_mb[c % 2],
                        msem[c % 2]).wait()
                    if c < SPC - 1:
                        @pl.when(cglob + 1 < nch)
                        def _():
                            pltpu.async_copy(
                                m_h.at[pl.ds(base + CHUNK, CHUNK)],
                                v_mb[(c + 1) % 2], msem[(c + 1) % 2])
                    v_m = v_mb[c % 2]

                    def group_body(q, carry2):
                        e0 = q * 16
                        dvec = v_dst[pl.ds(c * CHUNK + e0, 16)]
                        for j in range(16):
                            g = base + e0 + j
                            valid = jnp.logical_and(g >= b0, g < b1)
                            dl = jnp.where(valid, dvec[j] - v0, NV)
                            off = dl * Fh
                            e = e0 + j
                            for k in range(KF):
                                col = k * 16
                                mk = v_m[e, pl.ds(col, 16)]
                                so = off + col
                                plsc.addupdate(accs[0].at[pl.ds(so, 16)], mk)
                                plsc.addupdate(accs[1].at[pl.ds(so, 16)],
                                               mk * mk)
                                accs[2][pl.ds(so, 16)] = jnp.minimum(
                                    accs[2][pl.ds(so, 16)], mk)
                                accs[3][pl.ds(so, 16)] = jnp.maximum(
                                    accs[3][pl.ds(so, 16)], mk)
                            if with_cnt:
                                co = dl * 16
                                plsc.addupdate(accs[4].at[pl.ds(co, 16)],
                                               ones)
                        return carry2
                    lax.fori_loop(0, CHUNK // 16, group_body, 0)
            return carry
        lax.fori_loop(0, nsup, sup_body, 0)
        for j in range(4):
            pltpu.sync_copy(accs[j].at[pl.ds(0, NV * Fh)], outs[j].at[wid])
        if with_cnt:
            pltpu.sync_copy(accs[4].at[pl.ds(0, NV * 16)], outs[4].at[wid])

    fn = pl.kernel(body, out_type=tuple(out_type), mesh=mesh,
                   scratch_types=tuple(scratch), compiler_params=_SC_PARAMS)
    return fn(m_in, sdst, bnds)


def _full(shape):
    return pl.BlockSpec(shape, lambda i: tuple(0 for _ in shape))


def _rows(nb, f):
    return pl.BlockSpec((nb, f), lambda i: (i, 0))


def _m128_body(xd_ref, xs_ref, ea_ref, eW_ref, eb_ref, pW_ref, pb_ref,
               lo_ref, hi_ref):
    e = jnp.dot(ea_ref[...], eW_ref[...],
                preferred_element_type=jnp.float32) + eb_ref[...]
    h = jnp.concatenate([xd_ref[...], xs_ref[...], e], axis=1)
    m = jnp.dot(h, pW_ref[...], preferred_element_type=jnp.float32) \
        + pb_ref[...]
    lo_ref[...] = m[:, 0:64]
    hi_ref[...] = m[:, 64:128]


def _m128_call(xd_g, xs_g, sea_p, eW, eb, pW, pb):
    return pl.pallas_call(
        _m128_body,
        grid=(EPAD // EB,),
        in_specs=[_rows(EB, 128), _rows(EB, 128), _rows(EB, 4),
                  _full((4, 128)), _full((1, 128)), _full((384, 128)),
                  _full((1, 128))],
        out_specs=[_rows(EB, 64), _rows(EB, 64)],
        out_shape=[jax.ShapeDtypeStruct((EPAD, 64), jnp.float32)] * 2,
    )(xd_g, xs_g, sea_p, eW, eb.reshape(1, 128), pW, pb.reshape(1, 128))


def _m16_body(xd_ref, xs_ref, ea_ref, eW_ref, eb_ref, pW_ref, pb_ref, o_ref):
    e = jnp.dot(ea_ref[...], eW_ref[...],
                preferred_element_type=jnp.float32) + eb_ref[...]
    h = jnp.concatenate([xd_ref[..., 0:4], xs_ref[..., 0:4], e], axis=1)
    o_ref[...] = jnp.dot(h, pW_ref[...],
                         preferred_element_type=jnp.float32) + pb_ref[...]


def _m16_call(xd_g, xs_g, sea_p, eW, eb, pW_pad, pb_pad):
    return pl.pallas_call(
        _m16_body,
        grid=(EPAD // EB,),
        in_specs=[_rows(EB, 16), _rows(EB, 16), _rows(EB, 4),
                  _full((4, 4)), _full((1, 4)), _full((12, 16)),
                  _full((1, 16))],
        out_specs=_rows(EB, 16),
        out_shape=jax.ShapeDtypeStruct((EPAD, 16), jnp.float32),
    )(xd_g, xs_g, sea_p, eW, eb.reshape(1, 4), pW_pad, pb_pad.reshape(1, 16))


def _deg_body(cnt_ref, dcols_ref, sums_ref):
    i = pl.program_id(0)
    c = cnt_ref[...]
    deg = jnp.maximum(c, 1.0)
    logd = jnp.log(deg + 1.0)
    has = (c > 0).astype(jnp.float32)
    dcols_ref[...] = jnp.concatenate(
        [deg, logd, has, jnp.zeros_like(c)], axis=1)
    blk = jnp.concatenate(
        [jnp.sum(deg).reshape(1, 1), jnp.sum(logd).reshape(1, 1),
         jnp.zeros((1, 6), jnp.float32)], axis=1)

    @pl.when(i == 0)
    def _():
        sums_ref[...] = jnp.zeros_like(sums_ref)
    sums_ref[...] += blk


def _deg_call(cnt):
    return pl.pallas_call(
        _deg_body,
        grid=(GRID,),
        in_specs=[_rows(NB, 1)],
        out_specs=[_rows(NB, 4), _full((1, 8))],
        out_shape=[jax.ShapeDtypeStruct((NN, 4), jnp.float32),
                   jax.ShapeDtypeStruct((1, 8), jnp.float32)],
    )(cnt)


def _make_conv_body(f, nseg):
    def body(*refs):
        srefs = refs[:4 * nseg]
        (x_ref, dc_ref, sums_ref, oW_ref, ob_ref, lW_ref,
         lb_ref, o_ref, bn_ref) = refs[4 * nseg:]
        i = pl.program_id(0)
        dc = dc_ref[...]
        deg = dc[:, 0:1]
        logd = dc[:, 1:2]
        has = dc[:, 2:3]
        avg_lin = sums_ref[0, 0] / NN
        avg_log = sums_ref[0, 1] / NN

        def cat(j):
            v = jnp.concatenate([srefs[j * nseg + s][...]
                                 for s in range(nseg)], axis=1) if nseg > 1 \
                else srefs[j][...]
            return v[:, :f]
        ssum = cat(0)
        ssq = cat(1)
        mean = ssum / deg
        meansq = ssq / deg
        std = jnp.sqrt(jnp.maximum(meansq - mean * mean, 0.0) + 1e-5)
        mn = jnp.where(has > 0, cat(2), 0.0)
        mx = jnp.where(has > 0, cat(3), 0.0)
        agg = jnp.concatenate([mean, mn, mx, std], axis=1)
        s1 = logd / avg_log
        s2 = avg_log / logd
        s3 = deg / avg_lin
        h2 = jnp.concatenate([x_ref[...], agg, agg * s1, agg * s2, agg * s3],
                             axis=1)
        out = jnp.dot(h2, oW_ref[...],
                      preferred_element_type=jnp.float32) + ob_ref[...]
        out = jnp.dot(out, lW_ref[...],
                      preferred_element_type=jnp.float32) + lb_ref[...]
        o_ref[...] = out

        @pl.when(i == 0)
        def _():
            bn_ref[...] = jnp.zeros_like(bn_ref)
        bn_ref[...] += jnp.sum(out, 0, keepdims=True)
    return body


def _conv_call(stat_arrays, xc, dcols, sums, oW, ob, lW, lb, f, nseg):
    fp = stat_arrays[0].shape[1]
    in_specs = ([_rows(NB, fp)] * (4 * nseg)
                + [_rows(NB, xc.shape[1]), _rows(NB, 4), _full((1, 8)),
                   _full(oW.shape), _full((1, 128)),
                   _full((128, 128)), _full((1, 128))])
    return pl.pallas_call(
        _make_conv_body(f, nseg),
        grid=(GRID,),
        in_specs=in_specs,
        out_specs=[_rows(NB, 128), _full((1, 128))],
        out_shape=[jax.ShapeDtypeStruct((NN, 128), jnp.float32),
                   jax.ShapeDtypeStruct((1, 128), jnp.float32)],
    )(*stat_arrays, xc, dcols, sums, oW, ob.reshape(1, 128), lW,
      lb.reshape(1, 128))


def _var_body(o_ref, bn_ref, v_ref):
    i = pl.program_id(0)
    mu = bn_ref[...] / NN
    d = o_ref[...] - mu

    @pl.when(i == 0)
    def _():
        v_ref[...] = jnp.zeros_like(v_ref)
    v_ref[...] += jnp.sum(d * d, 0, keepdims=True)


def _var_call(out2, bn):
    return pl.pallas_call(
        _var_body,
        grid=(GRID,),
        in_specs=[_rows(NB, 128), _full((1, 128))],
        out_specs=_full((1, 128)),
        out_shape=jax.ShapeDtypeStruct((1, 128), jnp.float32),
    )(out2, bn)


def _fin_body(o_ref, bn_ref, vs_ref, g_ref, b_ref, xn_ref):
    mu = bn_ref[...] / NN
    var = vs_ref[...] / NN
    o = (o_ref[...] - mu) / jnp.sqrt(var + 1e-5) * g_ref[...] + b_ref[...]
    xn_ref[...] = jnp.where(o >= 0, o, 0.01 * o)


def _fin_call(out2, bn, vs, g, b):
    return pl.pallas_call(
        _fin_body,
        grid=(GRID,),
        in_specs=[_rows(NB, 128), _full((1, 128)), _full((1, 128)),
                  _full((1, 128)), _full((1, 128))],
        out_specs=_rows(NB, 128),
        out_shape=jax.ShapeDtypeStruct((NN, 128), jnp.float32),
    )(out2, bn, vs, g.reshape(1, 128), b.reshape(1, 128))


def _pool_body(x_ref, b_ref, w1_ref, b1_ref, w2_ref, b2_ref, w3_ref, b3_ref,
               out_ref, s_sum, s_max, s_cnt):
    i = pl.program_id(0)

    @pl.when(i == 0)
    def _():
        s_sum[...] = jnp.zeros_like(s_sum)
        s_max[...] = jnp.full_like(s_max, -BIGF)
        s_cnt[...] = jnp.zeros_like(s_cnt)

    xb = x_ref[...]
    bb = b_ref[...]
    for g in range(NGRP):
        mask = bb == g
        s = jnp.sum(jnp.where(mask, xb, 0.0), axis=0, keepdims=True)
        mx = jnp.max(jnp.where(mask, xb, -BIGF), axis=0, keepdims=True)
        cg = jnp.sum(mask.astype(jnp.float32))
        s_sum[g:g + 1, :] += s
        s_max[g:g + 1, :] = jnp.maximum(s_max[g:g + 1, :], mx)
        s_cnt[g:g + 1, :] += jnp.full((1, 128), 1.0, jnp.float32) * cg

    @pl.when(i == GRID - 1)
    def _():
        cnt = s_cnt[...]
        x1 = s_sum[...] / jnp.maximum(cnt, 1.0)
        x2 = jnp.where(cnt > 0, s_max[...], 0.0)
        z = jnp.concatenate([x1, x2], axis=1)
        z = jnp.dot(z, w1_ref[...],
                    preferred_element_type=jnp.float32) + b1_ref[...]
        z = jnp.where(z >= 0, z, 0.01 * z)
        z = jnp.dot(z, w2_ref[...],
                    preferred_element_type=jnp.float32) + b2_ref[...]
        z = jnp.where(z >= 0, z, 0.01 * z)
        o = jnp.dot(z, w3_ref[...],
                    preferred_element_type=jnp.float32) + b3_ref[...]
        out_ref[...] = o[:NGRP, :]


def _pool_call(xc, batch2, l1W, l1b, l2W, l2b, l3W, l3b):
    return pl.pallas_call(
        _pool_body,
        grid=(GRID,),
        in_specs=[_rows(NB, 128), _rows(NB, 1), _full((256, 128)),
                  _full((1, 128)), _full((128, 64)), _full((1, 64)),
                  _full((64, 1)), _full((1, 1))],
        out_specs=_full((NGRP, 1)),
        out_shape=jax.ShapeDtypeStruct((NGRP, 1), jnp.float32),
        scratch_shapes=[pltpu.VMEM((16, 128), jnp.float32),
                        pltpu.VMEM((16, 128), jnp.float32),
                        pltpu.VMEM((16, 128), jnp.float32)],
    )(xc, batch2, l1W, l1b.reshape(1, 128), l2W, l2b.reshape(1, 64), l3W,
      l3b.reshape(1, 1))


def kernel(x, edge_index, edge_attr, batch, params):
    src = edge_index[0]
    dst = edge_index[1]
    perm = jnp.argsort(dst)
    sdst = dst[perm]
    ssrc = src[perm]
    sea = edge_attr[perm]
    pad = EPAD - EE
    sdst_p = jnp.concatenate([sdst, jnp.zeros((pad,), jnp.int32)])
    ssrc_p = jnp.concatenate([ssrc, jnp.zeros((pad,), jnp.int32)])
    sea_p = jnp.concatenate([sea, jnp.zeros((pad, 4), jnp.float32)])
    ranges = jnp.minimum(jnp.arange(NW + 1, dtype=jnp.int32) * NV, NN)
    bnds = jnp.searchsorted(sdst, ranges, side='left').astype(jnp.int32)
    bnds_p = jnp.concatenate([bnds, jnp.full((48 - NW - 1,), EE, jnp.int32)])
    convs = params['convs']

    p0 = convs[0]
    xpad = jnp.pad(x, ((0, 0), (0, 12)))
    xd_g, xs_g = _sc_gather_call(xpad, sdst_p, ssrc_p, bnds_p, 16)
    pW_pad = jnp.pad(p0['pW'], ((0, 0), (0, 12)))
    pb_pad = jnp.pad(p0['pb'], (0, 12))
    m16 = _m16_call(xd_g, xs_g, sea_p, p0['eW'], p0['eb'], pW_pad, pb_pad)
    ssum, ssq, smn, smx, scnt = _sc_stats_call(m16, sdst_p, bnds_p, 16, True)
    cnt = scnt.reshape(NTOT, 16)[:NN, 0:1]
    dcols, sums = _deg_call(cnt)
    stat_arrays = [ssum.reshape(NTOT, 16), ssq.reshape(NTOT, 16),
                   smn.reshape(NTOT, 16), smx.reshape(NTOT, 16)]
    nseg = 1

    xc = x
    for li in range(6):
        p = convs[li]
        f = 4 if li == 0 else 128
        out2, bn = _conv_call(stat_arrays, xc, dcols, sums, p['oW'],
                              p['ob'], p['lW'], p['lb'], f, nseg)
        vs = _var_call(out2, bn)
        xc = _fin_call(out2, bn, vs, p['bn_g'], p['bn_b'])
        if li < 5:
            pn = convs[li + 1]
            xd_g, xs_g = _sc_gather_call(xc, sdst_p, ssrc_p, bnds_p, 128)
            m_lo, m_hi = _m128_call(xd_g, xs_g, sea_p, pn['eW'], pn['eb'],
                                    pn['pW'], pn['pb'])
            r_lo = _sc_stats_call(m_lo, sdst_p, bnds_p, 64, False)
            r_hi = _sc_stats_call(m_hi, sdst_p, bnds_p, 64, False)
            stat_arrays = []
            for a, b in zip(r_lo, r_hi):
                stat_arrays.append(a.reshape(NTOT, 64))
                stat_arrays.append(b.reshape(NTOT, 64))
            nseg = 2

    batch2 = batch.reshape(NN, 1)
    return _pool_call(xc, batch2, params['l1W'], params['l1b'],
                      params['l2W'], params['l2b'], params['l3W'],
                      params['l3b'])
